# Initial kernel scaffold; baseline (speedup 1.0000x reference)
import functools

import jax
import jax.numpy as jnp
from jax.experimental import pallas as pl
from jax.experimental.pallas import tpu as pltpu

_COEF = 0.01
_NEG_INF = float("-inf")


def _gate_kernel(x_ref, wg_ref, comb_ref, aux_ref, acc_ref, *, ntok, nexp):
    i = pl.program_id(0)
    x = x_ref[...]
    scores = jnp.dot(x, wg_ref[...].T,
                     precision=jax.lax.Precision.HIGHEST)
    ids = jax.lax.broadcasted_iota(jnp.int32, scores.shape, 1)
    s1 = jnp.max(scores, axis=-1, keepdims=True)
    i1 = jnp.min(jnp.where(scores == s1, ids, nexp), axis=-1, keepdims=True)
    hot1 = ids == i1
    masked = jnp.where(hot1, _NEG_INF, scores)
    s2 = jnp.max(masked, axis=-1, keepdims=True)
    i2 = jnp.min(jnp.where(masked == s2, ids, nexp), axis=-1, keepdims=True)
    hot2 = ids == i2
    d = jnp.exp(s2 - s1)
    w1 = 1.0 / (1.0 + d)
    w2 = d / (1.0 + d)
    comb = jnp.where(hot1, w1, 0.0) + jnp.where(hot2, w2, 0.0)
    comb_ref[...] = comb

    @pl.when(i == 0)
    def _():
        acc_ref[...] = jnp.zeros_like(acc_ref)

    acc_ref[...] += jnp.sum(comb, axis=0, keepdims=True)

    @pl.when(i == pl.num_programs(0) - 1)
    def _():
        usage = acc_ref[...] / ntok
        ent = -jnp.sum(usage * jnp.log(usage + 1e-10))
        aux_ref[0, 0] = _COEF * (1.0 - ent / jnp.log(float(nexp)))


def _expert_kernel(comb_ref, x_ref, w1_ref, b1_ref, w2_ref, b2_ref,
                   out_ref, acc_ref, *, nexp):
    e = pl.program_id(1)
    xb = x_ref[...].astype(jnp.bfloat16)
    w1 = w1_ref[0]
    h = jnp.dot(xb, w1.T, preferred_element_type=jnp.float32)
    h = h + b1_ref[...]
    h = jax.nn.gelu(h, approximate=False)
    y = jnp.dot(h.astype(jnp.bfloat16), w2_ref[0].T,
                preferred_element_type=jnp.float32)
    y = y + b2_ref[...]
    w = comb_ref[:, pl.ds(e, 1)]

    @pl.when(e == 0)
    def _():
        acc_ref[...] = jnp.zeros_like(acc_ref)

    acc_ref[...] += w * y

    @pl.when(e == nexp - 1)
    def _():
        out_ref[...] = acc_ref[...]


def kernel(x, Wg, W1, b1, W2, b2):
    B, T, C = x.shape
    E, HID, _ = W1.shape
    N = B * T
    x_flat = x.reshape(N, C)
    tblk = min(256, N)
    ngrid = N // tblk

    comb, aux = pl.pallas_call(
        functools.partial(_gate_kernel, ntok=N, nexp=E),
        grid=(ngrid,),
        in_specs=[
            pl.BlockSpec((tblk, C), lambda i: (i, 0)),
            pl.BlockSpec((E, C), lambda i: (0, 0)),
        ],
        out_specs=[
            pl.BlockSpec((tblk, E), lambda i: (i, 0)),
            pl.BlockSpec((1, 1), lambda i: (0, 0)),
        ],
        out_shape=[
            jax.ShapeDtypeStruct((N, E), jnp.float32),
            jax.ShapeDtypeStruct((1, 1), jnp.float32),
        ],
        scratch_shapes=[pltpu.VMEM((1, E), jnp.float32)],
    )(x_flat, Wg)

    W1b = W1.astype(jnp.bfloat16)
    W2b = W2.astype(jnp.bfloat16)

    y = pl.pallas_call(
        functools.partial(_expert_kernel, nexp=E),
        grid=(ngrid, E),
        in_specs=[
            pl.BlockSpec((tblk, E), lambda i, e: (i, 0)),
            pl.BlockSpec((tblk, C), lambda i, e: (i, 0)),
            pl.BlockSpec((1, HID, C), lambda i, e: (e, 0, 0)),
            pl.BlockSpec((1, HID), lambda i, e: (e, 0)),
            pl.BlockSpec((1, C, HID), lambda i, e: (e, 0, 0)),
            pl.BlockSpec((1, C), lambda i, e: (e, 0)),
        ],
        out_specs=pl.BlockSpec((tblk, C), lambda i, e: (i, 0)),
        out_shape=jax.ShapeDtypeStruct((N, C), jnp.float32),
        scratch_shapes=[pltpu.VMEM((tblk, C), jnp.float32)],
    )(comb, x_flat, W1b, b1, W2b, b2)

    return y.reshape(B, T, C), aux[0, 0]


# speedup vs baseline: 2.1744x; 2.1744x over previous
import functools

import jax
import jax.numpy as jnp
from jax.experimental import pallas as pl
from jax.experimental.pallas import tpu as pltpu

_COEF = 0.01
_NEG_INF = float("-inf")


def _gate_kernel(x_ref, wg_ref, comb_ref, aux_ref, acc_ref, *, ntok, nexp):
    i = pl.program_id(0)
    x = x_ref[...]
    scores = jnp.dot(x.astype(jnp.bfloat16), wg_ref[...].astype(jnp.bfloat16).T,
                     preferred_element_type=jnp.float32)
    ids = jax.lax.broadcasted_iota(jnp.int32, scores.shape, 1)
    s1 = jnp.max(scores, axis=-1, keepdims=True)
    i1 = jnp.min(jnp.where(scores == s1, ids, nexp), axis=-1, keepdims=True)
    hot1 = ids == i1
    masked = jnp.where(hot1, _NEG_INF, scores)
    s2 = jnp.max(masked, axis=-1, keepdims=True)
    i2 = jnp.min(jnp.where(masked == s2, ids, nexp), axis=-1, keepdims=True)
    hot2 = ids == i2
    d = jnp.exp(s2 - s1)
    w1 = 1.0 / (1.0 + d)
    w2 = d / (1.0 + d)
    comb = jnp.where(hot1, w1, 0.0) + jnp.where(hot2, w2, 0.0)
    comb_ref[...] = comb

    @pl.when(i == 0)
    def _():
        acc_ref[...] = jnp.zeros_like(acc_ref)

    acc_ref[...] += jnp.sum(comb, axis=0, keepdims=True)

    @pl.when(i == pl.num_programs(0) - 1)
    def _():
        usage = acc_ref[...] / ntok
        ent = -jnp.sum(usage * jnp.log(usage + 1e-10), axis=-1, keepdims=True)
        aux_ref[...] = _COEF * (1.0 - ent / jnp.log(float(nexp)))


def _expert_kernel(comb_ref, x_ref, w1_ref, b1_ref, w2_ref, b2_ref,
                   out_ref, acc_ref, *, nexp):
    e = pl.program_id(1)
    xb = x_ref[...].astype(jnp.bfloat16)
    w1 = w1_ref[0]
    h = jnp.dot(xb, w1.T, preferred_element_type=jnp.float32)
    h = h + b1_ref[0]
    h = 0.5 * h * (1.0 + jax.lax.erf(h * 0.7071067811865476))
    y = jnp.dot(h.astype(jnp.bfloat16), w2_ref[0].T,
                preferred_element_type=jnp.float32)
    y = y + b2_ref[0]
    comb = comb_ref[...]
    lane = jax.lax.broadcasted_iota(jnp.int32, comb.shape, 1)
    w = jnp.sum(jnp.where(lane == e, comb, 0.0), axis=-1, keepdims=True)

    @pl.when(e == 0)
    def _():
        acc_ref[...] = jnp.zeros_like(acc_ref)

    acc_ref[...] += w * y

    @pl.when(e == nexp - 1)
    def _():
        out_ref[...] = acc_ref[...]


def kernel(x, Wg, W1, b1, W2, b2):
    B, T, C = x.shape
    E, HID, _ = W1.shape
    N = B * T
    x_flat = x.reshape(N, C)
    tblk = min(256, N)
    ngrid = N // tblk

    comb, aux = pl.pallas_call(
        functools.partial(_gate_kernel, ntok=N, nexp=E),
        grid=(ngrid,),
        in_specs=[
            pl.BlockSpec((tblk, C), lambda i: (i, 0)),
            pl.BlockSpec((E, C), lambda i: (0, 0)),
        ],
        out_specs=[
            pl.BlockSpec((tblk, E), lambda i: (i, 0)),
            pl.BlockSpec((1, 1), lambda i: (0, 0)),
        ],
        out_shape=[
            jax.ShapeDtypeStruct((N, E), jnp.float32),
            jax.ShapeDtypeStruct((1, 1), jnp.float32),
        ],
        scratch_shapes=[pltpu.VMEM((1, E), jnp.float32)],
    )(x_flat, Wg)

    W1b = W1.astype(jnp.bfloat16)
    W2b = W2.astype(jnp.bfloat16)

    y = pl.pallas_call(
        functools.partial(_expert_kernel, nexp=E),
        grid=(ngrid, E),
        in_specs=[
            pl.BlockSpec((tblk, E), lambda i, e: (i, 0)),
            pl.BlockSpec((tblk, C), lambda i, e: (i, 0)),
            pl.BlockSpec((1, HID, C), lambda i, e: (e, 0, 0)),
            pl.BlockSpec((1, 1, HID), lambda i, e: (e, 0, 0)),
            pl.BlockSpec((1, C, HID), lambda i, e: (e, 0, 0)),
            pl.BlockSpec((1, 1, C), lambda i, e: (e, 0, 0)),
        ],
        out_specs=pl.BlockSpec((tblk, C), lambda i, e: (i, 0)),
        out_shape=jax.ShapeDtypeStruct((N, C), jnp.float32),
        scratch_shapes=[pltpu.VMEM((tblk, C), jnp.float32)],
    )(comb, x_flat, W1b, b1.reshape(E, 1, HID), W2b, b2.reshape(E, 1, C))

    return y.reshape(B, T, C), aux[0, 0]
